# Initial kernel scaffold; baseline (speedup 1.0000x reference)
import jax
import jax.numpy as jnp
from jax.experimental import pallas as pl


def kernel(atomic_numbers, edge_index, dij, uij, positions, W_sender, W_receiver, bessel_freqs, radial_weights):
    raise NotImplementedError("write your pallas kernel here")


# speedup vs baseline: 22.7658x; 22.7658x over previous
import functools
import math

import jax
import jax.numpy as jnp
from jax import lax
from jax.experimental import pallas as pl
from jax.experimental.pallas import tpu as pltpu
from jax.experimental.pallas import tpu_sc as plsc

_ZS = (1, 6, 7, 8, 9)
_CUTOFF = 4.0
_LXLYLZ = (
    (0, 0, 0),
    (1, 0, 0), (0, 1, 0), (0, 0, 1),
    (2, 0, 0), (1, 1, 0), (1, 0, 1), (0, 2, 0), (0, 1, 1), (0, 0, 2),
)
_L_OF = tuple(lx + ly + lz for (lx, ly, lz) in _LXLYLZ)
_NANG = len(_LXLYLZ)
_NRBF = 8
_NRA = _NRBF * _NANG
_NB = 12
_NAB = 4

_BE = 2048
_CH = 128
_TILES = 16
_NSC = 2


def _edge_feat_kernel(dux_ref, fr_ref, lx_ref, ly_ref, lz_ref, out_ref):
    d = dux_ref[:, 0:1]
    ux = dux_ref[:, 1:2]
    uy = dux_ref[:, 2:3]
    uz = dux_ref[:, 3:4]
    fr = fr_ref[...]
    lx = lx_ref[...]
    ly = ly_ref[...]
    lz = lz_ref[...]
    r = d * (1.0 / _CUTOFF)
    r2 = r * r
    r4 = r2 * r2
    r5 = r4 * r
    r6 = r5 * r
    r7 = r6 * r
    cut = 1.0 - 21.0 * r5 + 35.0 * r6 - 15.0 * r7
    cut = jnp.where(d < _CUTOFF, cut, 0.0)
    env = (math.sqrt(2.0 / _CUTOFF) * cut) / d
    rad = jnp.sin(d * fr) * env
    x = jnp.where(lx == 0.0, 1.0, jnp.where(lx == 1.0, ux, ux * ux))
    y = jnp.where(ly == 0.0, 1.0, jnp.where(ly == 1.0, uy, uy * uy))
    z = jnp.where(lz == 0.0, 1.0, jnp.where(lz == 1.0, uz, uz * uz))
    out_ref[...] = rad * (x * y * z)


def _make_sc_idx(e_pad, n_nodes):
    pt = e_pad // (_TILES * _NSC)
    nch = pt // _CH
    trash = n_nodes * 5

    mesh = plsc.VectorSubcoreMesh(core_axis_name="c", subcore_axis_name="s")

    @functools.partial(
        pl.kernel,
        out_type=jax.ShapeDtypeStruct((_TILES * _NSC, e_pad // (_TILES * _NSC)),
                                      jnp.int32),
        mesh=mesh,
        compiler_params=pltpu.CompilerParams(needs_layout_passes=False),
        scratch_types=[
            pltpu.VMEM((n_nodes,), jnp.int32),
            pltpu.VMEM((pt,), jnp.int32),
            pltpu.VMEM((pt,), jnp.int32),
            pltpu.VMEM((pt,), jnp.int32),
        ],
    )
    def sc_idx(send_hbm, recv_hbm, z_hbm, out_hbm, atb, recvb, idxb, sendb):
        c = lax.axis_index("c")
        s = lax.axis_index("s")
        wid = s * _NSC + c
        ebase = wid * pt
        lane = lax.iota(jnp.int32, 16)

        pltpu.sync_copy(z_hbm, atb)
        pltpu.sync_copy(recv_hbm.at[pl.ds(ebase, pt)], recvb)
        pltpu.sync_copy(send_hbm.at[pl.ds(ebase, pt)], sendb)

        @pl.loop(0, pt // 16)
        def _(k):
            sv = sendb[pl.ds(k * 16, 16)]
            zv = plsc.load_gather(atb, [sv])
            g = jnp.where(zv == 1, 0,
                jnp.where(zv == 6, 1,
                jnp.where(zv == 7, 2,
                jnp.where(zv == 8, 3, 4))))
            rv = recvb[pl.ds(k * 16, 16)]
            ok = (rv >= 0) & (rv < n_nodes)
            idx = jnp.where(ok, rv * 5 + g, trash + lane)
            idxb[pl.ds(k * 16, 16)] = idx

        pltpu.sync_copy(idxb, out_hbm.at[wid])
        return None

    return sc_idx


_ACC_ROWS = 50048


def _accum_kernel(idx_ref, g_ref, acc_ref):
    @pl.when(pl.program_id(0) == 0)
    def _():
        acc_ref[...] = jnp.zeros_like(acc_ref)

    def body(e, carry):
        row = idx_ref[0, 0, e]
        acc_ref[pl.ds(row, 1), :] = (acc_ref[pl.ds(row, 1), :]
                                     + g_ref[pl.ds(e, 1), :])
        return carry
    lax.fori_loop(0, g_ref.shape[0], body, 0, unroll=4)


def _node_kernel(k_ref, w_ref, z_ref, pat_ref, out_ref):
    kk = k_ref[...]
    w = w_ref[...]
    u = jnp.dot(kk, w, preferred_element_type=jnp.float32)
    z = z_ref[...]
    pat = pat_ref[...]
    er = jnp.zeros_like(u)
    for gi, zval in enumerate(_ZS):
        m = (z == zval).astype(jnp.float32)
        er = er + m * pat[gi:gi + 1, :]
    out_ref[...] = u * er


def kernel(atomic_numbers, edge_index, dij, uij, positions,
           W_sender, W_receiver, bessel_freqs, radial_weights):
    n_nodes = positions.shape[0]
    n_edges = edge_index.shape[1]

    fr80 = jnp.repeat(bessel_freqs.astype(jnp.float32), _NANG)[None, :]
    lx80 = jnp.tile(jnp.array([t[0] for t in _LXLYLZ], jnp.float32), _NRBF)[None, :]
    ly80 = jnp.tile(jnp.array([t[1] for t in _LXLYLZ], jnp.float32), _NRBF)[None, :]
    lz80 = jnp.tile(jnp.array([t[2] for t in _LXLYLZ], jnp.float32), _NRBF)[None, :]

    wper = radial_weights[jnp.array(_L_OF)]
    wperc = wper.reshape(_NANG, _NRBF, _NB, _NAB, _NAB)
    t6 = jnp.einsum('gi,arbij->grabij', W_sender.astype(jnp.float32), wperc)
    eye = jnp.eye(_NANG, dtype=jnp.float32)
    b7 = t6[:, :, None] * eye[None, None, :, :, None, None, None]
    bigw = b7.reshape(5 * _NRBF * _NANG, _NANG * _NB * _NAB * _NAB)
    pat = jnp.broadcast_to(
        W_receiver.astype(jnp.float32)[:, None, None, None, :],
        (5, _NANG, _NB, _NAB, _NAB)).reshape(5, 1920)

    chunk = _TILES * _CH
    e_pad = ((n_edges + chunk - 1) // chunk) * chunk
    pad = e_pad - n_edges
    d_p = jnp.concatenate([dij.astype(jnp.float32), jnp.ones((pad,), jnp.float32)])
    u_p = jnp.concatenate([uij.astype(jnp.float32), jnp.zeros((pad, 3), jnp.float32)])
    dux = jnp.concatenate([d_p[:, None], u_p], axis=1)
    send = jnp.concatenate([edge_index[0].astype(jnp.int32),
                            jnp.zeros((pad,), jnp.int32)])
    recv = jnp.concatenate([edge_index[1].astype(jnp.int32),
                            jnp.full((pad,), jnp.int32(2 ** 30))])
    zarr = atomic_numbers.astype(jnp.int32)

    g_edges = pl.pallas_call(
        _edge_feat_kernel,
        grid=(e_pad // _BE,),
        in_specs=[
            pl.BlockSpec((_BE, 4), lambda i: (i, 0)),
            pl.BlockSpec((1, _NRA), lambda i: (0, 0)),
            pl.BlockSpec((1, _NRA), lambda i: (0, 0)),
            pl.BlockSpec((1, _NRA), lambda i: (0, 0)),
            pl.BlockSpec((1, _NRA), lambda i: (0, 0)),
        ],
        out_specs=pl.BlockSpec((_BE, _NRA), lambda i: (i, 0)),
        out_shape=jax.ShapeDtypeStruct((e_pad, _NRA), jnp.float32),
    )(dux, fr80, lx80, ly80, lz80)

    if False:
        zv = zarr[send]
        gspec = jnp.where(zv == 1, 0, jnp.where(zv == 6, 1,
                jnp.where(zv == 7, 2, jnp.where(zv == 8, 3, 4))))
        okm = (recv >= 0) & (recv < n_nodes)
        idx = jnp.where(okm, recv * 5 + gspec, n_nodes * 5).astype(jnp.int32)
    else:
        sc_idx = _make_sc_idx(e_pad, n_nodes)
        idx = sc_idx(send, recv, zarr).reshape(e_pad)

    bbe = 2048
    idx3 = idx.reshape(e_pad // bbe, 1, bbe)
    acc = pl.pallas_call(
        _accum_kernel,
        grid=(e_pad // bbe,),
        in_specs=[
            pl.BlockSpec((1, 1, bbe), lambda i: (i, 0, 0),
                         memory_space=pltpu.SMEM),
            pl.BlockSpec((bbe, _NRA), lambda i: (i, 0)),
        ],
        out_specs=pl.BlockSpec((_ACC_ROWS, _NRA), lambda i: (0, 0)),
        out_shape=jax.ShapeDtypeStruct((_ACC_ROWS, _NRA), jnp.float32),
    )(idx3, g_edges)
    kmat = acc[:n_nodes * 5].reshape(n_nodes, 5 * _NRA)

    bn = 1000
    v = pl.pallas_call(
        _node_kernel,
        grid=(n_nodes // bn,),
        in_specs=[
            pl.BlockSpec((bn, 5 * _NRA), lambda i: (i, 0)),
            pl.BlockSpec((5 * _NRA, 1920), lambda i: (0, 0)),
            pl.BlockSpec((bn, 1), lambda i: (i, 0)),
            pl.BlockSpec((5, 1920), lambda i: (0, 0)),
        ],
        out_specs=pl.BlockSpec((bn, 1920), lambda i: (i, 0)),
        out_shape=jax.ShapeDtypeStruct((n_nodes, 1920), jnp.float32),
    )(kmat, bigw, zarr[:, None], pat)

    v3 = v.reshape(n_nodes, _NANG, _NB * _NAB * _NAB)
    out0 = v3[:, 0]
    out1 = jnp.stack([v3[:, 1], v3[:, 2], v3[:, 3]], axis=-1)
    comp = {(0, 0): 4, (0, 1): 5, (0, 2): 6,
            (1, 0): 5, (1, 1): 7, (1, 2): 8,
            (2, 0): 6, (2, 1): 8, (2, 2): 9}
    out2 = jnp.stack(
        [jnp.stack([v3[:, comp[(d1, d2)]] for d2 in range(3)], axis=-1)
         for d1 in range(3)], axis=-2)
    return (out0, out1, out2)
